# Initial kernel scaffold; baseline (speedup 1.0000x reference)
import jax
import jax.numpy as jnp
from jax.experimental import pallas as pl


def kernel(x, edge_index, W1, b1, W2, b2, Wl, bl):
    raise NotImplementedError("write your pallas kernel here")


# speedup vs baseline: 8.2405x; 8.2405x over previous
import jax
import jax.numpy as jnp
from jax import lax
from jax.experimental import pallas as pl
from jax.experimental.pallas import tpu as pltpu
from jax.experimental.pallas import tpu_sc as plsc

N_NODES = 10000
N_EDGES = 320000
D = 128

NC = 2
NS = 16
NW = NC * NS
K = 128
NCHUNK = 80
E_PAD = NW * NCHUNK * K
N_PAD = 10240
RPT = N_PAD // NS
TRASH = N_NODES

BLK = 1024
GRID = N_PAD // BLK


def _sc_mesh():
    return plsc.VectorSubcoreMesh(core_axis_name="c", subcore_axis_name="s",
                                  num_cores=NC, num_subcores=NS)


def _deg_body(dst_hbm, ones_hbm, zeros_hbm, deg_out, dst_idx, ones_v, acc):
    c = lax.axis_index("c")
    s = lax.axis_index("s")
    wid = s * NC + c
    base = s * RPT
    pltpu.sync_copy(ones_hbm, ones_v)
    pltpu.sync_copy(zeros_hbm, acc.at[pl.ds(base, RPT)])
    pltpu.sync_copy(dst_hbm.at[wid], dst_idx)
    plsc.subcore_barrier()

    @pl.loop(0, NCHUNK)
    def _(j):
        pltpu.sync_copy(ones_v, acc.at[dst_idx.at[j]], add=True)

    plsc.subcore_barrier()
    pltpu.sync_copy(acc.at[pl.ds(base, RPT)], deg_out.at[c, pl.ds(base, RPT)])


def _deg_call(dst_r, ones, zeros128):
    f = pl.kernel(
        _deg_body,
        out_type=jax.ShapeDtypeStruct((NC, N_PAD, D), jnp.float32),
        mesh=_sc_mesh(),
        scratch_types=[
            pltpu.VMEM((NCHUNK, K), jnp.int32),
            pltpu.VMEM((K, D), jnp.float32),
            pltpu.VMEM_SHARED((N_PAD, D), jnp.float32),
        ],
    )
    return f(dst_r, ones, zeros128)


def _msg_body(y_hbm, src_hbm, dst_hbm, zeros_hbm, out_hbm,
              src_idx, dst_idx, rows, acc, gsem):
    c = lax.axis_index("c")
    s = lax.axis_index("s")
    wid = s * NC + c
    base = s * RPT
    pltpu.sync_copy(zeros_hbm, acc.at[pl.ds(base, RPT)])
    pltpu.sync_copy(src_hbm.at[wid], src_idx)
    pltpu.sync_copy(dst_hbm.at[wid], dst_idx)
    plsc.subcore_barrier()

    @pl.loop(0, NCHUNK)
    def _(j):
        pltpu.async_copy(y_hbm.at[src_idx.at[j]], rows, gsem).wait()
        pltpu.sync_copy(rows, acc.at[dst_idx.at[j]], add=True)

    plsc.subcore_barrier()
    pltpu.sync_copy(acc.at[pl.ds(base, RPT)], out_hbm.at[c, pl.ds(base, RPT)])


def _msg_call(y, src_r, dst_r, zeros128):
    f = pl.kernel(
        _msg_body,
        out_type=jax.ShapeDtypeStruct((NC, N_PAD, D), jnp.float32),
        mesh=_sc_mesh(),
        scratch_types=[
            pltpu.VMEM((NCHUNK, K), jnp.int32),
            pltpu.VMEM((NCHUNK, K), jnp.int32),
            pltpu.VMEM((K, D), jnp.float32),
            pltpu.VMEM_SHARED((N_PAD, D), jnp.float32),
            pltpu.SemaphoreType.DMA,
        ],
    )
    return f(y, src_r, dst_r, zeros128)


def _dinv_blk(deg_ref):
    return lax.rsqrt(deg_ref[0, :, 0:1] + deg_ref[1, :, 0:1] + 1.0)


def _prep_body(x_ref, w_ref, deg_ref, y_ref):
    dinv = _dinv_blk(deg_ref)
    xw = jnp.dot(x_ref[...], w_ref[...], preferred_element_type=jnp.float32)
    y_ref[...] = xw * dinv


def _mid_body(deg_ref, acc_ref, y_ref, b_ref, w_ref, o_ref):
    dinv = _dinv_blk(deg_ref)
    a = acc_ref[0] + acc_ref[1] + y_ref[...]
    h = jnp.maximum(a * dinv + b_ref[...], 0.0)
    o_ref[...] = jnp.dot(h, w_ref[...], preferred_element_type=jnp.float32) * dinv


def _final_body(deg_ref, acc_ref, y_ref, b_ref, w_ref, bl_ref, o_ref):
    dinv = _dinv_blk(deg_ref)
    a = acc_ref[0] + acc_ref[1] + y_ref[...]
    h = jnp.maximum(a * dinv + b_ref[...], 0.0)
    o_ref[...] = (jnp.dot(h, w_ref[...], preferred_element_type=jnp.float32)
                  + bl_ref[...])


_ROWS = pl.BlockSpec((BLK, D), lambda i: (i, 0))
_PARTS_D = pl.BlockSpec((NC, BLK, D), lambda i: (0, i, 0))
_WMAT = pl.BlockSpec((D, D), lambda i: (0, 0))
_BROW = pl.BlockSpec((1, D), lambda i: (0, 0))
_OUT = jax.ShapeDtypeStruct((N_PAD, D), jnp.float32)


def _tc_prep(x, W1, deg_parts):
    return pl.pallas_call(
        _prep_body,
        grid=(GRID,),
        in_specs=[_ROWS, _WMAT, _PARTS_D],
        out_specs=_ROWS,
        out_shape=_OUT,
    )(x, W1, deg_parts)


def _tc_mid(deg_parts, acc_parts, y, b, W):
    return pl.pallas_call(
        _mid_body,
        grid=(GRID,),
        in_specs=[_PARTS_D, _PARTS_D, _ROWS, _BROW, _WMAT],
        out_specs=_ROWS,
        out_shape=_OUT,
    )(deg_parts, acc_parts, y, b, W)


def _tc_final(deg_parts, acc_parts, y, b, W, bl):
    return pl.pallas_call(
        _final_body,
        grid=(GRID,),
        in_specs=[_PARTS_D, _PARTS_D, _ROWS, _BROW, _WMAT, _BROW],
        out_specs=_ROWS,
        out_shape=_OUT,
    )(deg_parts, acc_parts, y, b, W, bl)


@jax.jit
def kernel(x, edge_index, W1, b1, W2, b2, Wl, bl):
    x_pad = jnp.zeros((N_PAD, D), jnp.float32).at[:N_NODES].set(x)
    ei = edge_index.astype(jnp.int32)
    pad = jnp.full((E_PAD - N_EDGES,), TRASH, jnp.int32)
    src_r = jnp.concatenate([ei[0], pad]).reshape(NW, NCHUNK, K)
    dst_r = jnp.concatenate([ei[1], pad]).reshape(NW, NCHUNK, K)

    ones = jnp.ones((K, D), jnp.float32)
    zeros128 = jnp.zeros((RPT, D), jnp.float32)
    b1r = b1.reshape(1, D)
    b2r = b2.reshape(1, D)
    blr = bl.reshape(1, D)

    deg_parts = _deg_call(dst_r, ones, zeros128)
    y1 = _tc_prep(x_pad, W1, deg_parts)
    acc1 = _msg_call(y1, src_r, dst_r, zeros128)
    y2 = _tc_mid(deg_parts, acc1, y1, b1r, W2)
    acc2 = _msg_call(y2, src_r, dst_r, zeros128)
    out = _tc_final(deg_parts, acc2, y2, b2r, Wl, blr)
    return out[:N_NODES]


# speedup vs baseline: 10.9618x; 1.3302x over previous
import jax
import jax.numpy as jnp
from jax import lax
from jax.experimental import pallas as pl
from jax.experimental.pallas import tpu as pltpu
from jax.experimental.pallas import tpu_sc as plsc

N_NODES = 10000
N_EDGES = 320000
D = 128

NC = 2
NS = 16
NW = NC * NS
K = 128
NCHUNK = 80
CH0 = 120
CH1 = 40
NCHUNKS_TOT = NS * (CH0 + CH1)
E_PAD = NCHUNKS_TOT * K
N_PAD = 10240
RPT = N_PAD // NS
TRASH = N_NODES

BLK = 1024
GRID = N_PAD // BLK


def _sc_mesh():
    return plsc.VectorSubcoreMesh(core_axis_name="c", subcore_axis_name="s",
                                  num_cores=NC, num_subcores=NS)


def _deg_body(dst_hbm, ones_hbm, zeros_hbm, deg_out, dst_idx, ones_v, acc):
    c = lax.axis_index("c")
    s = lax.axis_index("s")
    wid = s * NC + c
    base = s * RPT
    pltpu.sync_copy(ones_hbm, ones_v)
    pltpu.sync_copy(zeros_hbm, acc.at[pl.ds(base, RPT)])
    pltpu.sync_copy(dst_hbm.at[pl.ds(wid * NCHUNK, NCHUNK)], dst_idx)
    plsc.subcore_barrier()

    @pl.loop(0, NCHUNK)
    def _(j):
        pltpu.sync_copy(ones_v, acc.at[dst_idx.at[j]], add=True)

    plsc.subcore_barrier()
    pltpu.sync_copy(acc.at[pl.ds(base, RPT)], deg_out.at[c, pl.ds(base, RPT)])


def _deg_call(dst_r, ones, zeros128):
    f = pl.kernel(
        _deg_body,
        out_type=jax.ShapeDtypeStruct((NC, N_PAD, D), jnp.float32),
        mesh=_sc_mesh(),
        scratch_types=[
            pltpu.VMEM((NCHUNK, K), jnp.int32),
            pltpu.VMEM((K, D), jnp.float32),
            pltpu.VMEM_SHARED((N_PAD, D), jnp.float32),
        ],
    )
    return f(dst_r, ones, zeros128)


def _msg_body(y_hbm, src_hbm, dst_hbm, zeros_hbm, out_hbm,
              src_idx, dst_idx, rows, acc, gsem):
    c = lax.axis_index("c")
    s = lax.axis_index("s")
    base = s * RPT
    pltpu.sync_copy(zeros_hbm, acc.at[pl.ds(base, RPT)])

    @pl.when(c == 0)
    def _():
        pltpu.sync_copy(src_hbm.at[pl.ds(s * CH0, CH0)], src_idx)
        pltpu.sync_copy(dst_hbm.at[pl.ds(s * CH0, CH0)], dst_idx)

    @pl.when(c == 1)
    def _():
        off = NS * CH0 + s * CH1
        pltpu.sync_copy(src_hbm.at[pl.ds(off, CH1)], src_idx.at[pl.ds(0, CH1)])
        pltpu.sync_copy(dst_hbm.at[pl.ds(off, CH1)], dst_idx.at[pl.ds(0, CH1)])

    plsc.subcore_barrier()
    nch = jnp.where(c == 0, CH0, CH1)

    @pl.loop(0, nch)
    def _(j):
        pltpu.async_copy(y_hbm.at[src_idx.at[j]], rows, gsem).wait()
        pltpu.sync_copy(rows, acc.at[dst_idx.at[j]], add=True)

    plsc.subcore_barrier()
    pltpu.sync_copy(acc.at[pl.ds(base, RPT)], out_hbm.at[c, pl.ds(base, RPT)])


def _msg_call(y, src_r, dst_r, zeros128):
    f = pl.kernel(
        _msg_body,
        out_type=jax.ShapeDtypeStruct((NC, N_PAD, D), jnp.float32),
        mesh=_sc_mesh(),
        scratch_types=[
            pltpu.VMEM((CH0, K), jnp.int32),
            pltpu.VMEM((CH0, K), jnp.int32),
            pltpu.VMEM((K, D), jnp.float32),
            pltpu.VMEM_SHARED((N_PAD, D), jnp.float32),
            pltpu.SemaphoreType.DMA,
        ],
    )
    return f(y, src_r, dst_r, zeros128)


def _dinv_blk(deg_ref):
    return lax.rsqrt(deg_ref[0, :, 0:1] + deg_ref[1, :, 0:1] + 1.0)


def _prep_body(x_ref, w_ref, deg_ref, y_ref):
    dinv = _dinv_blk(deg_ref)
    xw = jnp.dot(x_ref[...], w_ref[...], preferred_element_type=jnp.float32)
    y_ref[...] = xw * dinv


def _mid_body(deg_ref, acc_ref, y_ref, b_ref, w_ref, o_ref):
    dinv = _dinv_blk(deg_ref)
    a = acc_ref[0] + acc_ref[1] + y_ref[...]
    h = jnp.maximum(a * dinv + b_ref[...], 0.0)
    o_ref[...] = jnp.dot(h, w_ref[...], preferred_element_type=jnp.float32) * dinv


def _final_body(deg_ref, acc_ref, y_ref, b_ref, w_ref, bl_ref, o_ref):
    dinv = _dinv_blk(deg_ref)
    a = acc_ref[0] + acc_ref[1] + y_ref[...]
    h = jnp.maximum(a * dinv + b_ref[...], 0.0)
    o_ref[...] = (jnp.dot(h, w_ref[...], preferred_element_type=jnp.float32)
                  + bl_ref[...])


_ROWS = pl.BlockSpec((BLK, D), lambda i: (i, 0))
_PARTS_D = pl.BlockSpec((NC, BLK, D), lambda i: (0, i, 0))
_WMAT = pl.BlockSpec((D, D), lambda i: (0, 0))
_BROW = pl.BlockSpec((1, D), lambda i: (0, 0))
_OUT = jax.ShapeDtypeStruct((N_PAD, D), jnp.float32)


def _tc_prep(x, W1, deg_parts):
    return pl.pallas_call(
        _prep_body,
        grid=(GRID,),
        in_specs=[_ROWS, _WMAT, _PARTS_D],
        out_specs=_ROWS,
        out_shape=_OUT,
    )(x, W1, deg_parts)


def _tc_mid(deg_parts, acc_parts, y, b, W):
    return pl.pallas_call(
        _mid_body,
        grid=(GRID,),
        in_specs=[_PARTS_D, _PARTS_D, _ROWS, _BROW, _WMAT],
        out_specs=_ROWS,
        out_shape=_OUT,
    )(deg_parts, acc_parts, y, b, W)


def _tc_final(deg_parts, acc_parts, y, b, W, bl):
    return pl.pallas_call(
        _final_body,
        grid=(GRID,),
        in_specs=[_PARTS_D, _PARTS_D, _ROWS, _BROW, _WMAT, _BROW],
        out_specs=_ROWS,
        out_shape=_OUT,
    )(deg_parts, acc_parts, y, b, W, bl)


@jax.jit
def kernel(x, edge_index, W1, b1, W2, b2, Wl, bl):
    x_pad = jnp.zeros((N_PAD, D), jnp.float32).at[:N_NODES].set(x)
    ei = edge_index.astype(jnp.int32)
    pad = jnp.full((E_PAD - N_EDGES,), TRASH, jnp.int32)
    src_r = jnp.concatenate([ei[0], pad]).reshape(NCHUNKS_TOT, K)
    dst_r = jnp.concatenate([ei[1], pad]).reshape(NCHUNKS_TOT, K)

    ones = jnp.ones((K, D), jnp.float32)
    zeros128 = jnp.zeros((RPT, D), jnp.float32)
    b1r = b1.reshape(1, D)
    b2r = b2.reshape(1, D)
    blr = bl.reshape(1, D)

    deg_parts = _deg_call(dst_r, ones, zeros128)
    y1 = _tc_prep(x_pad, W1, deg_parts)
    acc1 = _msg_call(y1, src_r, dst_r, zeros128)
    y2 = _tc_mid(deg_parts, acc1, y1, b1r, W2)
    acc2 = _msg_call(y2, src_r, dst_r, zeros128)
    out = _tc_final(deg_parts, acc2, y2, b2r, Wl, blr)
    return out[:N_NODES]


# speedup vs baseline: 11.5223x; 1.0511x over previous
import jax
import jax.numpy as jnp
from jax import lax
from jax.experimental import pallas as pl
from jax.experimental.pallas import tpu as pltpu
from jax.experimental.pallas import tpu_sc as plsc

N_NODES = 10000
N_EDGES = 320000
D = 128

NC = 2
NS = 16
NW = NC * NS
K = 128
NCHUNK = 80
CH0 = 120
CH1 = 40
NCHUNKS_TOT = NS * (CH0 + CH1)
E_PAD = NCHUNKS_TOT * K
N_PAD = 10240
RPT = N_PAD // NS
TRASH = N_NODES

BLK = 1024
GRID = N_PAD // BLK


def _sc_mesh():
    return plsc.VectorSubcoreMesh(core_axis_name="c", subcore_axis_name="s",
                                  num_cores=NC, num_subcores=NS)


def _deg_body(dst_hbm, ones_hbm, zeros_hbm, deg_out, dst_idx, ones_v, acc):
    c = lax.axis_index("c")
    s = lax.axis_index("s")
    wid = s * NC + c
    base = s * RPT
    pltpu.sync_copy(ones_hbm, ones_v)
    pltpu.sync_copy(zeros_hbm, acc.at[pl.ds(base, RPT)])
    pltpu.sync_copy(dst_hbm.at[pl.ds(wid * NCHUNK, NCHUNK)], dst_idx)
    plsc.subcore_barrier()

    @pl.loop(0, NCHUNK)
    def _(j):
        pltpu.sync_copy(ones_v, acc.at[dst_idx.at[j]], add=True)

    plsc.subcore_barrier()
    pltpu.sync_copy(acc.at[pl.ds(base, RPT)], deg_out.at[c, pl.ds(base, RPT)])


def _deg_call(dst_r, ones, zeros128):
    f = pl.kernel(
        _deg_body,
        out_type=jax.ShapeDtypeStruct((NC, N_PAD, D), jnp.float32),
        mesh=_sc_mesh(),
        scratch_types=[
            pltpu.VMEM((NCHUNK, K), jnp.int32),
            pltpu.VMEM((K, D), jnp.float32),
            pltpu.VMEM_SHARED((N_PAD, D), jnp.float32),
        ],
    )
    return f(dst_r, ones, zeros128)


NBUF = 2


def _msg_body(y_hbm, src_hbm, dst_hbm, zeros_hbm, out_hbm,
              sidx, didx, rows, acc, gsems, isems):
    c = lax.axis_index("c")
    s = lax.axis_index("s")
    base = s * RPT
    pltpu.sync_copy(zeros_hbm, acc.at[pl.ds(base, RPT)])

    start = jnp.where(c == 0, s * CH0, NS * CH0 + s * CH1)
    ngrp = jnp.where(c == 0, CH0 // NBUF, CH1 // NBUF)

    def load_idx(g, p, sync):
        off = start + g * NBUF
        if sync:
            pltpu.sync_copy(src_hbm.at[pl.ds(off, NBUF)], sidx.at[p])
            pltpu.sync_copy(dst_hbm.at[pl.ds(off, NBUF)], didx.at[p])
        else:
            pltpu.async_copy(src_hbm.at[pl.ds(off, NBUF)], sidx.at[p],
                             isems.at[p])
            pltpu.async_copy(dst_hbm.at[pl.ds(off, NBUF)], didx.at[p],
                             isems.at[p])

    def wait_idx(p):
        pltpu.make_async_copy(src_hbm.at[pl.ds(0, NBUF)], sidx.at[p],
                              isems.at[p]).wait()
        pltpu.make_async_copy(dst_hbm.at[pl.ds(0, NBUF)], didx.at[p],
                              isems.at[p]).wait()

    def gather(p, b):
        pltpu.async_copy(y_hbm.at[sidx.at[p, b]], rows.at[b], gsems.at[b])

    def wait_gather(p, b):
        pltpu.make_async_copy(y_hbm.at[sidx.at[p, b]], rows.at[b],
                              gsems.at[b]).wait()

    load_idx(0, 0, True)
    for b in range(NBUF):
        gather(0, b)
    load_idx(1, 1, False)
    plsc.subcore_barrier()

    def group(g, p):
        pnext = 1 - p
        @pl.when(g + 1 < ngrp)
        def _():
            wait_idx(pnext)

        for b in range(NBUF):
            wait_gather(p, b)
            pltpu.sync_copy(rows.at[b], acc.at[didx.at[p, b]], add=True)

            @pl.when(g + 1 < ngrp)
            def _():
                gather(pnext, b)

        @pl.when(g + 2 < ngrp)
        def _():
            load_idx(g + 2, p, False)

    @pl.loop(0, ngrp // 2)
    def _(u):
        for parity in range(2):
            group(u * 2 + parity, parity)

    plsc.subcore_barrier()
    pltpu.sync_copy(acc.at[pl.ds(base, RPT)], out_hbm.at[c, pl.ds(base, RPT)])


def _msg_call(y, src_r, dst_r, zeros128):
    f = pl.kernel(
        _msg_body,
        out_type=jax.ShapeDtypeStruct((NC, N_PAD, D), jnp.float32),
        mesh=_sc_mesh(),
        scratch_types=[
            pltpu.VMEM((2, NBUF, K), jnp.int32),
            pltpu.VMEM((2, NBUF, K), jnp.int32),
            pltpu.VMEM((NBUF, K, D), jnp.float32),
            pltpu.VMEM_SHARED((N_PAD, D), jnp.float32),
            pltpu.SemaphoreType.DMA((NBUF,)),
            pltpu.SemaphoreType.DMA((2,)),
        ],
    )
    return f(y, src_r, dst_r, zeros128)


def _dinv_blk(deg_ref):
    return lax.rsqrt(deg_ref[0, :, 0:1] + deg_ref[1, :, 0:1] + 1.0)


def _prep_body(x_ref, w_ref, deg_ref, y_ref):
    dinv = _dinv_blk(deg_ref)
    xw = jnp.dot(x_ref[...], w_ref[...], preferred_element_type=jnp.float32)
    y_ref[...] = xw * dinv


def _mid_body(deg_ref, acc_ref, y_ref, b_ref, w_ref, o_ref):
    dinv = _dinv_blk(deg_ref)
    a = acc_ref[0] + acc_ref[1] + y_ref[...]
    h = jnp.maximum(a * dinv + b_ref[...], 0.0)
    o_ref[...] = jnp.dot(h, w_ref[...], preferred_element_type=jnp.float32) * dinv


def _final_body(deg_ref, acc_ref, y_ref, b_ref, w_ref, bl_ref, o_ref):
    dinv = _dinv_blk(deg_ref)
    a = acc_ref[0] + acc_ref[1] + y_ref[...]
    h = jnp.maximum(a * dinv + b_ref[...], 0.0)
    o_ref[...] = (jnp.dot(h, w_ref[...], preferred_element_type=jnp.float32)
                  + bl_ref[...])


_ROWS = pl.BlockSpec((BLK, D), lambda i: (i, 0))
_PARTS_D = pl.BlockSpec((NC, BLK, D), lambda i: (0, i, 0))
_WMAT = pl.BlockSpec((D, D), lambda i: (0, 0))
_BROW = pl.BlockSpec((1, D), lambda i: (0, 0))
_OUT = jax.ShapeDtypeStruct((N_PAD, D), jnp.float32)


def _tc_prep(x, W1, deg_parts):
    return pl.pallas_call(
        _prep_body,
        grid=(GRID,),
        in_specs=[_ROWS, _WMAT, _PARTS_D],
        out_specs=_ROWS,
        out_shape=_OUT,
    )(x, W1, deg_parts)


def _tc_mid(deg_parts, acc_parts, y, b, W):
    return pl.pallas_call(
        _mid_body,
        grid=(GRID,),
        in_specs=[_PARTS_D, _PARTS_D, _ROWS, _BROW, _WMAT],
        out_specs=_ROWS,
        out_shape=_OUT,
    )(deg_parts, acc_parts, y, b, W)


def _tc_final(deg_parts, acc_parts, y, b, W, bl):
    return pl.pallas_call(
        _final_body,
        grid=(GRID,),
        in_specs=[_PARTS_D, _PARTS_D, _ROWS, _BROW, _WMAT, _BROW],
        out_specs=_ROWS,
        out_shape=_OUT,
    )(deg_parts, acc_parts, y, b, W, bl)


@jax.jit
def kernel(x, edge_index, W1, b1, W2, b2, Wl, bl):
    x_pad = jnp.zeros((N_PAD, D), jnp.float32).at[:N_NODES].set(x)
    ei = edge_index.astype(jnp.int32)
    pad = jnp.full((E_PAD - N_EDGES,), TRASH, jnp.int32)
    src_r = jnp.concatenate([ei[0], pad]).reshape(NCHUNKS_TOT, K)
    dst_r = jnp.concatenate([ei[1], pad]).reshape(NCHUNKS_TOT, K)

    ones = jnp.ones((K, D), jnp.float32)
    zeros128 = jnp.zeros((RPT, D), jnp.float32)
    b1r = b1.reshape(1, D)
    b2r = b2.reshape(1, D)
    blr = bl.reshape(1, D)

    deg_parts = _deg_call(dst_r, ones, zeros128)
    y1 = _tc_prep(x_pad, W1, deg_parts)
    acc1 = _msg_call(y1, src_r, dst_r, zeros128)
    y2 = _tc_mid(deg_parts, acc1, y1, b1r, W2)
    acc2 = _msg_call(y2, src_r, dst_r, zeros128)
    out = _tc_final(deg_parts, acc2, y2, b2r, Wl, blr)
    return out[:N_NODES]
